# Initial kernel scaffold; baseline (speedup 1.0000x reference)
import jax
import jax.numpy as jnp
from jax.experimental import pallas as pl


def kernel(x, weight):
    raise NotImplementedError("write your pallas kernel here")


# speedup vs baseline: 1.1013x; 1.1013x over previous
import functools

import jax
import jax.numpy as jnp
from jax import lax
from jax.experimental import pallas as pl
from jax.experimental.pallas import tpu as pltpu
from jax.experimental.pallas import tpu_sc as plsc

D_MODEL = 32


@functools.partial(jax.jit, static_argnames=())
def _embed(flat_idx, weight):
    B = flat_idx.shape[0]
    info = plsc.get_sparse_core_info()
    NW = info.num_cores * info.num_subcores
    b_per_w = B // NW
    CH = 1600
    n_steps = b_per_w // CH

    mesh = plsc.VectorSubcoreMesh(core_axis_name="c", subcore_axis_name="s")

    @functools.partial(
        pl.kernel,
        mesh=mesh,
        out_type=jax.ShapeDtypeStruct((B, D_MODEL), jnp.float32),
        scratch_types=[
            pltpu.VMEM((CH,), jnp.int32),
            pltpu.VMEM((CH, D_MODEL), jnp.float32),
            pltpu.SemaphoreType.DMA,
        ],
        compiler_params=pltpu.CompilerParams(use_tc_tiling_on_sc=False),
    )
    def k(idx_hbm, table_hbm, out_hbm, idx_v, rows_v, sem):
        wid = lax.axis_index("s") * info.num_cores + lax.axis_index("c")
        base = wid * b_per_w

        def body(i, carry):
            off = base + i * CH
            pltpu.sync_copy(idx_hbm.at[pl.ds(off, CH)], idx_v)
            pltpu.async_copy(table_hbm.at[idx_v], rows_v, sem).wait()
            pltpu.sync_copy(rows_v, out_hbm.at[pl.ds(off, CH)])
            return carry

        lax.fori_loop(0, n_steps, body, 0)

    return k(flat_idx, weight)


def kernel(x, weight):
    flat = x.reshape(-1).astype(jnp.int32)
    out = _embed(flat, weight)
    return out.reshape(x.shape + (weight.shape[1],))


# speedup vs baseline: 1.1112x; 1.0090x over previous
import functools

import jax
import jax.numpy as jnp
from jax import lax
from jax.experimental import pallas as pl
from jax.experimental.pallas import tpu as pltpu
from jax.experimental.pallas import tpu_sc as plsc

D_MODEL = 32


@jax.jit
def _embed(flat_idx, weight):
    B = flat_idx.shape[0]
    info = plsc.get_sparse_core_info()
    NW = info.num_cores * info.num_subcores
    b_per_w = B // NW
    CH = 1600
    n_steps = b_per_w // CH

    mesh = plsc.VectorSubcoreMesh(core_axis_name="c", subcore_axis_name="s")

    @functools.partial(
        pl.kernel,
        mesh=mesh,
        out_type=jax.ShapeDtypeStruct((B, D_MODEL), jnp.float32),
        scratch_types=[
            pltpu.VMEM((CH,), jnp.int32),
            pltpu.VMEM((CH,), jnp.int32),
            pltpu.VMEM((CH, D_MODEL), jnp.float32),
            pltpu.VMEM((CH, D_MODEL), jnp.float32),
            pltpu.SemaphoreType.DMA,
            pltpu.SemaphoreType.DMA,
        ],
        compiler_params=pltpu.CompilerParams(use_tc_tiling_on_sc=False),
    )
    def k(idx_hbm, table_hbm, out_hbm, idx0, idx1, rows0, rows1, gsem, osem):
        wid = lax.axis_index("s") * info.num_cores + lax.axis_index("c")
        base = wid * b_per_w
        idx_v = [idx0, idx1]
        rows_v = [rows0, rows1]

        gather = [None] * n_steps
        wb = [None] * n_steps

        pltpu.sync_copy(idx_hbm.at[pl.ds(base, CH)], idx_v[0])
        gather[0] = pltpu.async_copy(table_hbm.at[idx_v[0]], rows_v[0], gsem)
        for i in range(n_steps):
            b = i % 2
            if i + 1 < n_steps:
                nb = (i + 1) % 2
                pltpu.sync_copy(
                    idx_hbm.at[pl.ds(base + (i + 1) * CH, CH)], idx_v[nb]
                )
                if i >= 1:
                    wb[i - 1].wait()
                gather[i + 1] = pltpu.async_copy(
                    table_hbm.at[idx_v[nb]], rows_v[nb], gsem
                )
            gather[i].wait()
            wb[i] = pltpu.async_copy(
                rows_v[b], out_hbm.at[pl.ds(base + i * CH, CH)], osem
            )
        wb[n_steps - 2].wait()
        wb[n_steps - 1].wait()

    return k(flat_idx, weight)


def kernel(x, weight):
    flat = x.reshape(-1).astype(jnp.int32)
    out = _embed(flat, weight)
    return out.reshape(x.shape + (weight.shape[1],))


# speedup vs baseline: 1.9381x; 1.7441x over previous
import functools

import jax
import jax.numpy as jnp
from jax import lax
from jax.experimental import pallas as pl
from jax.experimental.pallas import tpu as pltpu
from jax.experimental.pallas import tpu_sc as plsc

D_MODEL = 32


@jax.jit
def _embed_t(xt, weight):
    S, BT = xt.shape
    D = D_MODEL
    info = plsc.get_sparse_core_info()
    NW = info.num_cores * info.num_subcores
    BW = BT // NW

    mesh = plsc.VectorSubcoreMesh(core_axis_name="c", subcore_axis_name="s")

    @functools.partial(
        pl.kernel,
        mesh=mesh,
        out_type=jax.ShapeDtypeStruct((S, BT, D), jnp.float32),
        scratch_types=[
            pltpu.VMEM((S, BW), jnp.int32),
            pltpu.VMEM((BW, D), jnp.float32),
            pltpu.VMEM((BW, D), jnp.float32),
            pltpu.SemaphoreType.DMA,
            pltpu.SemaphoreType.DMA,
        ],
        compiler_params=pltpu.CompilerParams(use_tc_tiling_on_sc=False),
    )
    def k(xt_hbm, table_hbm, out_hbm, idx_all, rows0, rows1, gsem, osem):
        wid = lax.axis_index("s") * info.num_cores + lax.axis_index("c")
        b0 = wid * BW
        rows = [rows0, rows1]

        pltpu.sync_copy(xt_hbm.at[:, pl.ds(b0, BW)], idx_all)

        gathers = [None] * S
        wbs = [None] * S
        gathers[0] = pltpu.async_copy(
            table_hbm.at[idx_all.at[0]], rows[0], gsem
        )
        for s in range(S):
            p = s % 2
            if s + 1 < S:
                if s >= 1:
                    wbs[s - 1].wait()
                gathers[s + 1] = pltpu.async_copy(
                    table_hbm.at[idx_all.at[s + 1]], rows[(s + 1) % 2], gsem
                )
            gathers[s].wait()
            wbs[s] = pltpu.async_copy(
                rows[p], out_hbm.at[s, pl.ds(b0, BW), :], osem
            )
        wbs[S - 2].wait()
        wbs[S - 1].wait()

    return k(xt, weight)


def kernel(x, weight):
    out_t = _embed_t(x.T, weight)
    return out_t.transpose(1, 0, 2)
